# baseline (speedup 1.0000x reference)
import functools

import jax
import jax.numpy as jnp
import numpy as np
from jax import lax
from jax.experimental import pallas as pl
from jax.experimental.pallas import tpu as pltpu
from jax.experimental.pallas import tpu_sc as plsc

NC = 2
NS = 16
NW = NC * NS
CHUNK = 128


def _dense1(x, Wc, b1):
    N, D = x.shape
    H = Wc.shape[1] // 2

    def body(x_ref, w_ref, b_ref, y1_ref, r1_ref):
        y = jnp.dot(x_ref[...], w_ref[...], preferred_element_type=jnp.float32)
        y1_ref[...] = y[:, :H]
        r1_ref[...] = y[:, H:] + b_ref[...]

    return pl.pallas_call(
        body,
        out_shape=(jax.ShapeDtypeStruct((N, H), jnp.float32),
                   jax.ShapeDtypeStruct((N, H), jnp.float32)),
    )(x, Wc, b1.reshape(1, H))


def _dense2(p0, p1, c0, c1, r1, W2, b2):
    N, H = p0.shape

    def body(p0_ref, p1_ref, c0_ref, c1_ref, r1_ref, w_ref, b2_ref,
             y2p_ref, r2_ref, cnt_ref):
        cnt = jnp.maximum(c0_ref[...] + c1_ref[...], 1.0)[:, 0:1]
        h = jnp.maximum((p0_ref[...] + p1_ref[...]) / cnt + r1_ref[...], 0.0)
        y = jnp.dot(h, w_ref[...], preferred_element_type=jnp.float32)
        y2p_ref[...] = jnp.concatenate(
            [y[:, 0:1], jnp.zeros((N, 15), jnp.float32)], axis=1)
        r2_ref[...] = y[:, 1:2] + b2_ref[...]
        cnt_ref[...] = cnt

    return pl.pallas_call(
        body,
        out_shape=(jax.ShapeDtypeStruct((N, 16), jnp.float32),
                   jax.ShapeDtypeStruct((N, 1), jnp.float32),
                   jax.ShapeDtypeStruct((N, 1), jnp.float32)),
    )(p0, p1, c0, c1, r1, W2, b2.reshape(1, 1))


def _final(q0, q1, cnt, r2):
    N = q0.shape[0]

    def body(q0_ref, q1_ref, cnt_ref, r2_ref, o_ref):
        s = (q0_ref[...] + q1_ref[...])[:, 0:1]
        o_ref[...] = s / cnt_ref[...] + r2_ref[...]

    return pl.pallas_call(
        body,
        out_shape=jax.ShapeDtypeStruct((N, 1), jnp.float32),
    )(q0, q1, cnt, r2)


def _fill(ref, rows, width, value):
    v = jnp.full((16,), value, jnp.float32)

    def fi(i, carry):
        for jj in range(width // 16):
            ref[i, pl.ds(jj * 16, 16)] = v
        return carry

    lax.fori_loop(0, rows, fi, 0)


def _segsum_sc(table, srcp, dstp, npad, with_counts):
    _, NCH, _ = srcp.shape
    W = table.shape[1]
    RPT = npad // NS
    mesh = plsc.VectorSubcoreMesh(core_axis_name="c", subcore_axis_name="s")

    z_rows = jnp.zeros((RPT, W), jnp.float32)
    inputs = [table, srcp, dstp, z_rows]
    out_type = [pltpu.HBM((NC, npad, W), jnp.float32)]
    scratch = [
        pltpu.VMEM((NCH, CHUNK), jnp.int32),
        pltpu.VMEM((NCH, CHUNK), jnp.int32),
        pltpu.VMEM((CHUNK, W), jnp.float32),
        pltpu.VMEM((RPT, W), jnp.float32),
        pltpu.VMEM_SHARED((npad, W), jnp.float32),
        pltpu.SemaphoreType.DMA,
    ]
    if with_counts:
        out_type.append(pltpu.HBM((NC, npad, 16), jnp.float32))
        inputs += [jnp.ones((CHUNK, 16), jnp.float32),
                   jnp.zeros((RPT, 16), jnp.float32)]
        scratch += [
            pltpu.VMEM((CHUNK, 16), jnp.float32),
            pltpu.VMEM((RPT, 16), jnp.float32),
            pltpu.VMEM_SHARED((npad, 16), jnp.float32),
        ]

    def body(*refs):
        if with_counts:
            (tab_h, srcp_h, dstp_h, z_h, ones_h, zc_h, part_o, cnt_o,
             src_v, dst_v, rows_v, stage_v, acc_s, sem0,
             ones_v, cstage_v, cacc_s) = refs
        else:
            (tab_h, srcp_h, dstp_h, z_h, part_o,
             src_v, dst_v, rows_v, stage_v, acc_s, sem0) = refs
        c = lax.axis_index("c")
        s = lax.axis_index("s")
        w = c * NS + s
        r0 = s * RPT
        pltpu.sync_copy(z_h, stage_v)
        pltpu.sync_copy(stage_v, acc_s.at[pl.ds(r0, RPT)])
        if with_counts:
            pltpu.sync_copy(ones_h, ones_v)
            pltpu.sync_copy(zc_h, cstage_v)
            pltpu.sync_copy(cstage_v, cacc_s.at[pl.ds(r0, RPT)])
        pltpu.sync_copy(srcp_h.at[w], src_v)
        pltpu.sync_copy(dstp_h.at[w], dst_v)
        plsc.subcore_barrier()

        def step(j, carry):
            pltpu.async_copy(tab_h.at[src_v.at[j]], rows_v, sem0).wait()
            pltpu.sync_copy(rows_v, acc_s.at[dst_v.at[j]], add=True)
            if with_counts:
                pltpu.sync_copy(ones_v, cacc_s.at[dst_v.at[j]], add=True)
            return carry

        lax.fori_loop(0, NCH, step, 0)
        plsc.subcore_barrier()
        pltpu.sync_copy(acc_s.at[pl.ds(r0, RPT)], stage_v)
        pltpu.sync_copy(stage_v, part_o.at[c, pl.ds(r0, RPT)])
        if with_counts:
            pltpu.sync_copy(cacc_s.at[pl.ds(r0, RPT)], cstage_v)
            pltpu.sync_copy(cstage_v, cnt_o.at[c, pl.ds(r0, RPT)])

    fn = pl.kernel(body, out_type=tuple(out_type), mesh=mesh,
                   scratch_types=tuple(scratch),
                   compiler_params=pltpu.CompilerParams(
                       use_tc_tiling_on_sc=False))
    return fn(*inputs)


def kernel(x, e, W1l, W1r, b1, W2l, W2r, b2):
    N, D = x.shape
    E = e.shape[1]

    ept = -(-E // NW)
    nch = -(-ept // CHUNK)
    nch = -(-nch // 2) * 2
    e_pad = NW * nch * CHUNK
    rpt = -(-(N + 1) // NS)
    rpt = -(-rpt // 2) * 2
    npad = rpt * NS

    src = e[0].astype(jnp.int32)
    dst = e[1].astype(jnp.int32)
    src = jnp.pad(src, (0, e_pad - E)).reshape(NW, nch, CHUNK)
    pos = jnp.arange(e_pad, dtype=jnp.int32)
    dst = jnp.where(pos < E, jnp.pad(dst, (0, e_pad - E)),
                    N + (pos % (npad - N))).reshape(NW, nch, CHUNK)

    Wc = jnp.concatenate([W1l, W1r], axis=1)
    y1, r1 = _dense1(x, Wc, b1)

    part, cnt = _segsum_sc(y1, src, dst, npad, with_counts=True)

    W2 = jnp.concatenate([W2l, W2r], axis=1)
    y2p, r2, cntm = _dense2(part[0, :N], part[1, :N],
                            cnt[0, :N], cnt[1, :N], r1, W2, b2)

    (part2,) = _segsum_sc(y2p, src, dst, npad, with_counts=False)

    return _final(part2[0, :N], part2[1, :N], cntm, r2)


# speedup vs baseline: 1.3039x; 1.3039x over previous
import functools

import jax
import jax.numpy as jnp
import numpy as np
from jax import lax
from jax.experimental import pallas as pl
from jax.experimental.pallas import tpu as pltpu
from jax.experimental.pallas import tpu_sc as plsc

NC = 2
NS = 16
NW = NC * NS
CHUNK = 128


def _dense1(x, Wc, b1):
    N, D = x.shape
    H = Wc.shape[1] // 2

    def body(x_ref, w_ref, b_ref, y1_ref, r1_ref):
        y = jnp.dot(x_ref[...], w_ref[...], preferred_element_type=jnp.float32)
        y1_ref[...] = y[:, :H]
        r1_ref[...] = y[:, H:] + b_ref[...]

    return pl.pallas_call(
        body,
        out_shape=(jax.ShapeDtypeStruct((N, H), jnp.float32),
                   jax.ShapeDtypeStruct((N, H), jnp.float32)),
    )(x, Wc, b1.reshape(1, H))


def _dense2(p0, p1, c0, c1, r1, W2, b2):
    N, H = p0.shape

    def body(p0_ref, p1_ref, c0_ref, c1_ref, r1_ref, w_ref, b2_ref,
             y2p_ref, r2_ref, cnt_ref):
        cnt = jnp.maximum(c0_ref[...] + c1_ref[...], 1.0)[:, 0:1]
        h = jnp.maximum((p0_ref[...] + p1_ref[...]) / cnt + r1_ref[...], 0.0)
        y = jnp.dot(h, w_ref[...], preferred_element_type=jnp.float32)
        y2p_ref[...] = jnp.concatenate(
            [y[:, 0:1], jnp.zeros((N, 15), jnp.float32)], axis=1)
        r2_ref[...] = y[:, 1:2] + b2_ref[...]
        cnt_ref[...] = cnt

    return pl.pallas_call(
        body,
        out_shape=(jax.ShapeDtypeStruct((N, 16), jnp.float32),
                   jax.ShapeDtypeStruct((N, 1), jnp.float32),
                   jax.ShapeDtypeStruct((N, 1), jnp.float32)),
    )(p0, p1, c0, c1, r1, W2, b2.reshape(1, 1))


def _final(q0, q1, cnt, r2):
    N = q0.shape[0]

    def body(q0_ref, q1_ref, cnt_ref, r2_ref, o_ref):
        s = (q0_ref[...] + q1_ref[...])[:, 0:1]
        o_ref[...] = s / cnt_ref[...] + r2_ref[...]

    return pl.pallas_call(
        body,
        out_shape=jax.ShapeDtypeStruct((N, 1), jnp.float32),
    )(q0, q1, cnt, r2)


def _fill(ref, rows, width, value):
    v = jnp.full((16,), value, jnp.float32)

    def fi(i, carry):
        for jj in range(width // 16):
            ref[i, pl.ds(jj * 16, 16)] = v
        return carry

    lax.fori_loop(0, rows, fi, 0)


def _segsum_sc(table, srcp, dstp, npad, with_counts):
    _, NCH, _ = srcp.shape
    W = table.shape[1]
    RPT = npad // NS
    mesh = plsc.VectorSubcoreMesh(core_axis_name="c", subcore_axis_name="s")

    z_rows = jnp.zeros((RPT, W), jnp.float32)
    inputs = [table, srcp, dstp, z_rows]
    out_type = [pltpu.HBM((NC, npad, W), jnp.float32)]
    scratch = [
        pltpu.VMEM((NCH, CHUNK), jnp.int32),
        pltpu.VMEM((NCH, CHUNK), jnp.int32),
        pltpu.VMEM((CHUNK, W), jnp.float32),
        pltpu.VMEM((RPT, W), jnp.float32),
        pltpu.VMEM_SHARED((npad, W), jnp.float32),
        pltpu.SemaphoreType.DMA,
    ]
    if with_counts:
        out_type.append(pltpu.HBM((NC, npad, 16), jnp.float32))
        inputs += [jnp.ones((CHUNK, 16), jnp.float32),
                   jnp.zeros((RPT, 16), jnp.float32)]
        scratch += [
            pltpu.VMEM((CHUNK, 16), jnp.float32),
            pltpu.VMEM((RPT, 16), jnp.float32),
            pltpu.VMEM_SHARED((npad, 16), jnp.float32),
        ]

    def body(*refs):
        if with_counts:
            (tab_h, srcp_h, dstp_h, z_h, ones_h, zc_h, part_o, cnt_o,
             src_v, dst_v, rows_v, stage_v, acc_s, sem0,
             ones_v, cstage_v, cacc_s) = refs
        else:
            (tab_h, srcp_h, dstp_h, z_h, part_o,
             src_v, dst_v, rows_v, stage_v, acc_s, sem0) = refs
        c = lax.axis_index("c")
        s = lax.axis_index("s")
        w = c * NS + s
        r0 = s * RPT
        pltpu.sync_copy(z_h, stage_v)
        pltpu.sync_copy(stage_v, acc_s.at[pl.ds(r0, RPT)])
        if with_counts:
            pltpu.sync_copy(ones_h, ones_v)
            pltpu.sync_copy(zc_h, cstage_v)
            pltpu.sync_copy(cstage_v, cacc_s.at[pl.ds(r0, RPT)])
        pltpu.sync_copy(srcp_h.at[w], src_v)
        pltpu.sync_copy(dstp_h.at[w], dst_v)
        plsc.subcore_barrier()

        def step(j, carry):
            pltpu.async_copy(tab_h.at[src_v.at[j]], rows_v, sem0).wait()
            pltpu.sync_copy(rows_v, acc_s.at[dst_v.at[j]], add=True)
            if with_counts:
                pltpu.sync_copy(ones_v, cacc_s.at[dst_v.at[j]], add=True)
            return carry

        lax.fori_loop(0, NCH, step, 0)
        plsc.subcore_barrier()
        pltpu.sync_copy(acc_s.at[pl.ds(r0, RPT)], stage_v)
        pltpu.sync_copy(stage_v, part_o.at[c, pl.ds(r0, RPT)])
        if with_counts:
            pltpu.sync_copy(cacc_s.at[pl.ds(r0, RPT)], cstage_v)
            pltpu.sync_copy(cstage_v, cnt_o.at[c, pl.ds(r0, RPT)])

    fn = pl.kernel(body, out_type=tuple(out_type), mesh=mesh,
                   scratch_types=tuple(scratch),
                   compiler_params=pltpu.CompilerParams(
                       use_tc_tiling_on_sc=False))
    return fn(*inputs)


def kernel(x, e, W1l, W1r, b1, W2l, W2r, b2):
    N, D = x.shape
    E = e.shape[1]

    ept = -(-E // NW)
    nch = -(-ept // CHUNK)
    e_pad = NW * nch * CHUNK
    rpt = -(-(N + 1) // NS)
    rpt = -(-rpt // 2) * 2
    npad = rpt * NS

    src = e[0].astype(jnp.int32)
    dst = e[1].astype(jnp.int32)
    src = jnp.pad(src, (0, e_pad - E)).reshape(NW, nch, CHUNK)
    dst = jnp.pad(dst, (0, e_pad - E), constant_values=N).reshape(NW, nch, CHUNK)

    Wc = jnp.concatenate([W1l, W1r], axis=1)
    y1, r1 = _dense1(x, Wc, b1)

    part, cnt = _segsum_sc(y1, src, dst, npad, with_counts=True)

    W2 = jnp.concatenate([W2l, W2r], axis=1)
    y2p, r2, cntm = _dense2(part[0, :N], part[1, :N],
                            cnt[0, :N], cnt[1, :N], r1, W2, b2)

    (part2,) = _segsum_sc(y2p, src, dst, npad, with_counts=False)

    return _final(part2[0, :N], part2[1, :N], cntm, r2)
